# Initial kernel scaffold; baseline (speedup 1.0000x reference)
import jax
import jax.numpy as jnp
from jax.experimental import pallas as pl


def kernel(feat, edge_index, node_key_w, node_key_b, node_query, edge_key_w, edge_key_b):
    raise NotImplementedError("write your pallas kernel here")


# speedup vs baseline: 11.8421x; 11.8421x over previous
import functools

import jax
import jax.numpy as jnp
from jax import lax
from jax.experimental import pallas as pl
from jax.experimental.pallas import tpu as pltpu
from jax.experimental.pallas import tpu_sc as plsc

N = 10000
E = 320000
D = 128
NPAD = 10240
NC, NS, L = 2, 16, 16
NW = NC * NS
EPW = E // NW
BK = 80
NBLK = EPW // BK
NROW = NPAD // NS


def _g_body(feat_ref, nkw_ref, nkb_ref, g_ref):
    g_ref[...] = feat_ref[...] * (1.0 + nkw_ref[...]) + nkb_ref[...]


def _build_g(feat, nkw, nkb):
    return pl.pallas_call(
        _g_body,
        grid=(25,),
        in_specs=[pl.BlockSpec((400, D), lambda i: (i, 0))] * 3,
        out_specs=pl.BlockSpec((400, D), lambda i: (i, 0)),
        out_shape=jax.ShapeDtypeStruct((N, D), jnp.float32),
    )(feat, nkw, nkb)


def _edge_body(feat_hbm, g_hbm, q_hbm, ekw_hbm, ekb_hbm, src_hbm, dst_hbm,
               ex_out, s2_out,
               src_c, dst_c, ekb_c, frows, grows, qrows, ekwrows,
               s_loc, accA, accB, accQ, exblk, s_stage, stmp, sacc, sem):
    cid = lax.axis_index("c")
    sid = lax.axis_index("s")
    wid = cid * NS + sid
    ebase = wid * EPW

    pltpu.sync_copy(src_hbm.at[pl.ds(ebase, EPW)], src_c)
    pltpu.sync_copy(dst_hbm.at[pl.ds(ebase, EPW)], dst_c)
    pltpu.sync_copy(ekb_hbm.at[pl.ds(ebase, EPW)], ekb_c)

    zero16 = jnp.zeros((L,), jnp.float32)

    def zloop(i, _):
        s_loc[pl.ds(i * L, L)] = zero16
        return 0

    lax.fori_loop(0, NPAD // L, zloop, 0)

    rowi = lax.iota(jnp.int32, L)

    def blk(b, _):
        bb = b * BK
        h1 = pltpu.async_copy(feat_hbm.at[src_c.at[pl.ds(bb, BK)]], frows, sem)
        h2 = pltpu.async_copy(g_hbm.at[src_c.at[pl.ds(bb, BK)]], grows, sem)
        h3 = pltpu.async_copy(q_hbm.at[dst_c.at[pl.ds(bb, BK)]], qrows, sem)
        h4 = pltpu.async_copy(ekw_hbm.at[pl.ds(ebase + bb, BK)], ekwrows, sem)
        h1.wait()
        h2.wait()
        h3.wait()
        h4.wait()

        def sub(t, _):
            be = t * L
            for e in range(L):
                r = be + e
                fa = frows[r, pl.ds(0, L)] * ekwrows[r, pl.ds(0, L)]
                qv = qrows[r, pl.ds(0, L)]
                fb = grows[r, pl.ds(0, L)] * qv
                fq = qv
                for j in range(1, D // L):
                    fa = fa + frows[r, pl.ds(j * L, L)] * ekwrows[r, pl.ds(j * L, L)]
                    qv = qrows[r, pl.ds(j * L, L)]
                    fb = fb + grows[r, pl.ds(j * L, L)] * qv
                    fq = fq + qv
                accA[pl.ds(e * (L + 1), L)] = fa
                accB[pl.ds(e * (L + 1), L)] = fb
                accQ[pl.ds(e * (L + 1), L)] = fq
            av = zero16
            bv = zero16
            qs = zero16
            coli = rowi * (L + 1)
            for l in range(L):
                li = coli + l
                av = av + plsc.load_gather(accA, [li])
                bv = bv + plsc.load_gather(accB, [li])
                qs = qs + plsc.load_gather(accQ, [li])
            ekb_v = ekb_c[pl.ds(bb + be, L)]
            dst_v = dst_c[pl.ds(bb + be, L)]
            logit = (av + ekb_v) * qs + bv
            exv = jnp.exp(logit)
            exblk[pl.ds(be, L)] = exv
            plsc.addupdate_scatter(s_loc, [dst_v], exv)
            return 0

        lax.fori_loop(0, BK // L, sub, 0)
        pltpu.sync_copy(exblk, ex_out.at[pl.ds(ebase + bb, BK)])
        return 0

    lax.fori_loop(0, NBLK, blk, 0)

    pltpu.sync_copy(s_loc, s_stage.at[sid])
    plsc.subcore_barrier()
    nb = sid * NROW

    def srow(i, _):
        sacc[pl.ds(i * L, L)] = zero16
        return 0

    lax.fori_loop(0, NROW // L, srow, 0)

    def comb(t, _):
        pltpu.sync_copy(s_stage.at[t, pl.ds(nb, NROW)], stmp)

        def addv(i, _):
            sacc[pl.ds(i * L, L)] = sacc[pl.ds(i * L, L)] + stmp[pl.ds(i * L, L)]
            return 0

        lax.fori_loop(0, NROW // L, addv, 0)
        return 0

    lax.fori_loop(0, NS, comb, 0)
    pltpu.sync_copy(sacc, s2_out.at[cid, pl.ds(nb, NROW)])


def _norm_body(ex_hbm, s2_hbm, dst_hbm, alpha_out,
               s_tot, stmp, dst_c, ex_c, al_c, sem):
    cid = lax.axis_index("c")
    sid = lax.axis_index("s")
    wid = cid * NS + sid
    ebase = wid * EPW

    pltpu.sync_copy(s2_hbm.at[0], s_tot)
    pltpu.sync_copy(s2_hbm.at[1], stmp)

    def addv(i, _):
        s_tot[pl.ds(i * L, L)] = s_tot[pl.ds(i * L, L)] + stmp[pl.ds(i * L, L)]
        return 0

    lax.fori_loop(0, NPAD // L, addv, 0)

    pltpu.sync_copy(dst_hbm.at[pl.ds(ebase, EPW)], dst_c)
    pltpu.sync_copy(ex_hbm.at[pl.ds(ebase, EPW)], ex_c)

    one16 = jnp.ones((L,), jnp.float32)

    def grp(i, _):
        dst_v = dst_c[pl.ds(i * L, L)]
        sv = plsc.load_gather(s_tot, [dst_v])
        exv = ex_c[pl.ds(i * L, L)]
        al_c[pl.ds(i * L, L)] = exv / jnp.where(sv > 0.0, sv, one16)
        return 0

    lax.fori_loop(0, EPW // L, grp, 0)
    pltpu.sync_copy(al_c, alpha_out.at[pl.ds(ebase, EPW)])


def kernel(feat, edge_index, node_key_w, node_key_b, node_query, edge_key_w, edge_key_b):
    src = edge_index[0]
    dst = edge_index[1]
    nkw = node_key_w[:, :, 0]
    ekb = edge_key_b[:, 0]

    g = _build_g(feat, nkw, node_key_b)

    mesh = plsc.VectorSubcoreMesh(core_axis_name="c", subcore_axis_name="s")

    sc_params = pltpu.CompilerParams(needs_layout_passes=False)

    edge_k = pl.kernel(
        _edge_body,
        compiler_params=sc_params,
        out_type=(
            jax.ShapeDtypeStruct((E,), jnp.float32),
            jax.ShapeDtypeStruct((NC, NPAD), jnp.float32),
        ),
        mesh=mesh,
        scratch_types=[
            pltpu.VMEM((EPW,), jnp.int32),
            pltpu.VMEM((EPW,), jnp.int32),
            pltpu.VMEM((EPW,), jnp.float32),
            pltpu.VMEM((BK, D), jnp.float32),
            pltpu.VMEM((BK, D), jnp.float32),
            pltpu.VMEM((BK, D), jnp.float32),
            pltpu.VMEM((BK, D), jnp.float32),
            pltpu.VMEM((NPAD,), jnp.float32),
            pltpu.VMEM((L * (L + 1),), jnp.float32),
            pltpu.VMEM((L * (L + 1),), jnp.float32),
            pltpu.VMEM((L * (L + 1),), jnp.float32),
            pltpu.VMEM((BK,), jnp.float32),
            pltpu.VMEM_SHARED((NS, NPAD), jnp.float32),
            pltpu.VMEM((NROW,), jnp.float32),
            pltpu.VMEM((NROW,), jnp.float32),
            pltpu.SemaphoreType.DMA,
        ],
    )
    ex, s2 = edge_k(feat, g, node_query, edge_key_w, ekb, src, dst)

    norm_k = pl.kernel(
        _norm_body,
        compiler_params=sc_params,
        out_type=jax.ShapeDtypeStruct((E,), jnp.float32),
        mesh=mesh,
        scratch_types=[
            pltpu.VMEM((NPAD,), jnp.float32),
            pltpu.VMEM((NPAD,), jnp.float32),
            pltpu.VMEM((EPW,), jnp.int32),
            pltpu.VMEM((EPW,), jnp.float32),
            pltpu.VMEM((EPW,), jnp.float32),
            pltpu.SemaphoreType.DMA,
        ],
    )
    alpha = norm_k(ex, s2, dst)
    return alpha[:, None]
